# baseline (speedup 1.0000x reference)
import functools

import jax
import jax.numpy as jnp
from jax import lax
from jax.experimental import pallas as pl
from jax.experimental.pallas import tpu as pltpu
from jax.experimental.pallas import tpu_sc as plsc

_LANES = 16


@functools.lru_cache(maxsize=None)
def _build(batch, dim):
    info = plsc.get_sparse_core_info()
    nw = info.num_cores * info.num_subcores
    nc = info.num_cores
    rows_per_line = 128 // dim
    b_per_w = batch // nw
    groups = b_per_w // _LANES
    out_lines_per_w = b_per_w * dim // 128
    mesh = plsc.VectorSubcoreMesh(core_axis_name="c", subcore_axis_name="s")

    @functools.partial(
        pl.kernel,
        mesh=mesh,
        out_type=jax.ShapeDtypeStruct((batch * dim // 128, 128), jnp.float32),
        scratch_types=[
            pltpu.VMEM((b_per_w,), jnp.int32),
            pltpu.VMEM((b_per_w,), jnp.int32),
            pltpu.VMEM((b_per_w, 128), jnp.float32),
            pltpu.VMEM((out_lines_per_w, 128), jnp.float32),
            pltpu.SemaphoreType.DMA,
        ],
        compiler_params=pltpu.CompilerParams(needs_layout_passes=False),
    )
    def gather_kernel(idx_hbm, table_hbm, out_hbm, idx_v, lines_v, rows_v,
                      out_v, sem):
        wid = lax.axis_index("s") * nc + lax.axis_index("c")
        base = wid * b_per_w
        pltpu.sync_copy(idx_hbm.at[pl.ds(base, b_per_w)], idx_v)

        def line_body(g, _):
            sl = pl.ds(g * _LANES, _LANES)
            lines_v[sl] = lax.shift_right_logical(idx_v[sl], 3)
            return _

        lax.fori_loop(0, groups, line_body, None, unroll=4)

        pltpu.async_copy(table_hbm.at[lines_v], rows_v, sem).wait()

        iota = lax.broadcasted_iota(jnp.int32, (_LANES,), 0)

        def sel_body(g, _):
            sl = pl.ds(g * _LANES, _LANES)
            idx16 = idx_v[sl]
            sub16 = (idx16 & (rows_per_line - 1)) * dim
            row16 = g * _LANES + iota
            flat_base = row16 * dim
            for j in range(dim):
                vals = plsc.load_gather(rows_v, [row16, sub16 + j])
                f = flat_base + j
                plsc.store_scatter(
                    out_v, [lax.shift_right_logical(f, 7), f & 127], vals)
            return _

        lax.fori_loop(0, groups, sel_body, None)

        pltpu.sync_copy(out_v, out_hbm.at[pl.ds(wid * out_lines_per_w,
                                                out_lines_per_w)])

    return gather_kernel


def kernel(ixs, z):
    n, dim = z.shape
    batch = ixs.shape[0]
    z_lines = z.reshape(n * dim // 128, 128)
    out_lines = _build(batch, dim)(ixs, z_lines)
    return out_lines.reshape(batch, dim)


# speedup vs baseline: 6.2261x; 6.2261x over previous
import functools

import jax
import jax.numpy as jnp
from jax import lax
from jax.experimental import pallas as pl
from jax.experimental.pallas import tpu as pltpu
from jax.experimental.pallas import tpu_sc as plsc

_WAVE = 16
_LANES = 128


@functools.lru_cache(maxsize=None)
def _build(batch, dim):
    info = plsc.get_sparse_core_info()
    nc = info.num_cores
    nw = nc * info.num_subcores
    chunk = batch // nw
    n_waves = chunk // _WAVE
    assert n_waves % 2 == 0
    mesh = plsc.VectorSubcoreMesh(core_axis_name="c", subcore_axis_name="s")

    @functools.partial(
        pl.kernel,
        mesh=mesh,
        out_type=jax.ShapeDtypeStruct((dim, batch), jnp.float32),
        scratch_types=[
            pltpu.VMEM((chunk,), jnp.int32),
            pltpu.VMEM((dim, 2 * _WAVE * _LANES), jnp.float32),
            pltpu.VMEM((dim, chunk), jnp.float32),
            pltpu.SemaphoreType.DMA,
            pltpu.SemaphoreType.DMA,
        ],
        compiler_params=pltpu.CompilerParams(needs_layout_passes=False),
    )
    def gather_kernel(idx_hbm, zt_hbm, out_hbm, idx_v, tiles_v, cols_v,
                      sem0, sem1):
        wid = lax.axis_index("s") * nc + lax.axis_index("c")
        base = pl.multiple_of(wid * chunk, _LANES)
        pltpu.sync_copy(idx_hbm.at[pl.ds(base, chunk)], idx_v)

        iota = lax.broadcasted_iota(jnp.int32, (_WAVE,), 0)
        sems = (sem0, sem1)

        def fire(g, buf):
            idx16 = idx_v[pl.ds(g * _WAVE, _WAVE)]
            for k in range(_WAVE):
                lt = pl.multiple_of(
                    lax.shift_right_logical(idx16[k], 7) * _LANES, _LANES)
                pltpu.async_copy(
                    zt_hbm.at[:, pl.ds(lt, _LANES)],
                    tiles_v.at[:, pl.ds((buf * _WAVE + k) * _LANES, _LANES)],
                    sems[buf])

        def drain_extract(g, buf):
            pltpu.make_async_copy(
                zt_hbm.at[:, pl.ds(0, _WAVE * _LANES)],
                tiles_v.at[:, pl.ds(buf * _WAVE * _LANES, _WAVE * _LANES)],
                sems[buf]).wait()
            idx16 = idx_v[pl.ds(g * _WAVE, _WAVE)]
            col16 = (buf * _WAVE + iota) * _LANES + (idx16 & (_LANES - 1))
            for j in range(dim):
                row_j = jnp.full((_WAVE,), j, jnp.int32)
                vals = plsc.load_gather(tiles_v, [row_j, col16])
                cols_v[j, pl.ds(g * _WAVE, _WAVE)] = vals

        fire(0, 0)

        def pair_body(p, _):
            g0 = 2 * p
            fire(g0 + 1, 1)
            drain_extract(g0, 0)

            @pl.when(g0 + 2 < n_waves)
            def _fire_next():
                fire(g0 + 2, 0)

            drain_extract(g0 + 1, 1)
            return _

        lax.fori_loop(0, n_waves // 2, pair_body, None)
        pltpu.sync_copy(cols_v, out_hbm.at[:, pl.ds(base, chunk)])

    return gather_kernel


def kernel(ixs, z):
    n, dim = z.shape
    batch = ixs.shape[0]
    out_t = _build(batch, dim)(ixs, z.T)
    return out_t.T


# speedup vs baseline: 6.2870x; 1.0098x over previous
import functools

import jax
import jax.numpy as jnp
from jax import lax
from jax.experimental import pallas as pl
from jax.experimental.pallas import tpu as pltpu
from jax.experimental.pallas import tpu_sc as plsc

_WAVE = 16
_LANES = 128


@functools.lru_cache(maxsize=None)
def _build(batch, dim):
    info = plsc.get_sparse_core_info()
    nc = info.num_cores
    nw = nc * info.num_subcores
    chunk = batch // nw
    n_waves = chunk // _WAVE
    assert n_waves % 2 == 0
    mesh = plsc.VectorSubcoreMesh(core_axis_name="c", subcore_axis_name="s")

    @functools.partial(
        pl.kernel,
        mesh=mesh,
        out_type=jax.ShapeDtypeStruct((dim, batch), jnp.float32),
        scratch_types=[
            pltpu.VMEM((chunk,), jnp.int32),
            pltpu.VMEM((dim, 2 * _WAVE * _LANES), jnp.float32),
            pltpu.VMEM((dim, chunk), jnp.float32),
            pltpu.SemaphoreType.DMA,
            pltpu.SemaphoreType.DMA,
        ],
        compiler_params=pltpu.CompilerParams(
            needs_layout_passes=False, skip_device_barrier=True),
    )
    def gather_kernel(idx_hbm, zt_hbm, out_hbm, idx_v, tiles_v, cols_v,
                      sem0, sem1):
        wid = lax.axis_index("s") * nc + lax.axis_index("c")
        base = pl.multiple_of(wid * chunk, _LANES)
        pltpu.sync_copy(idx_hbm.at[pl.ds(base, chunk)], idx_v)

        iota = lax.broadcasted_iota(jnp.int32, (_WAVE,), 0)
        sems = (sem0, sem1)

        def fire(g, buf):
            idx16 = idx_v[pl.ds(g * _WAVE, _WAVE)]
            for k in range(_WAVE):
                lt = pl.multiple_of(
                    lax.shift_right_logical(idx16[k], 7) * _LANES, _LANES)
                pltpu.async_copy(
                    zt_hbm.at[:, pl.ds(lt, _LANES)],
                    tiles_v.at[:, pl.ds((buf * _WAVE + k) * _LANES, _LANES)],
                    sems[buf])

        def drain_extract(g, buf):
            pltpu.make_async_copy(
                zt_hbm.at[:, pl.ds(0, _WAVE * _LANES)],
                tiles_v.at[:, pl.ds(buf * _WAVE * _LANES, _WAVE * _LANES)],
                sems[buf]).wait()
            idx16 = idx_v[pl.ds(g * _WAVE, _WAVE)]
            col16 = (buf * _WAVE + iota) * _LANES + (idx16 & (_LANES - 1))
            for j in range(dim):
                row_j = jnp.full((_WAVE,), j, jnp.int32)
                vals = plsc.load_gather(tiles_v, [row_j, col16])
                cols_v[j, pl.ds(g * _WAVE, _WAVE)] = vals

        fire(0, 0)

        def pair_body(p, _):
            g0 = 2 * p
            fire(g0 + 1, 1)
            drain_extract(g0, 0)

            @pl.when(g0 + 2 < n_waves)
            def _fire_next():
                fire(g0 + 2, 0)

            drain_extract(g0 + 1, 1)
            return _

        lax.fori_loop(0, n_waves // 2, pair_body, None)
        pltpu.sync_copy(cols_v, out_hbm.at[:, pl.ds(base, chunk)])

    return gather_kernel


def kernel(ixs, z):
    n, dim = z.shape
    batch = ixs.shape[0]
    out_t = _build(batch, dim)(ixs, z.T)
    return out_t.T


# speedup vs baseline: 6.7613x; 1.0754x over previous
import functools

import jax
import jax.numpy as jnp
from jax import lax
from jax.experimental import pallas as pl
from jax.experimental.pallas import tpu as pltpu
from jax.experimental.pallas import tpu_sc as plsc

_WAVE = 16
_NBUF = 3
_LANES = 128


@functools.lru_cache(maxsize=None)
def _build(batch, dim):
    info = plsc.get_sparse_core_info()
    nc = info.num_cores
    nw = nc * info.num_subcores
    chunk = batch // nw
    n_waves = chunk // _WAVE
    assert n_waves % 2 == 0 and n_waves >= 2 * _NBUF
    mesh = plsc.VectorSubcoreMesh(core_axis_name="c", subcore_axis_name="s")

    @functools.partial(
        pl.kernel,
        mesh=mesh,
        out_type=jax.ShapeDtypeStruct((dim, batch), jnp.float32),
        scratch_types=[
            pltpu.VMEM((chunk,), jnp.int32),
            pltpu.VMEM((dim, _NBUF * _WAVE * _LANES), jnp.float32),
            pltpu.VMEM((dim, chunk), jnp.float32),
            pltpu.SemaphoreType.DMA,
            pltpu.SemaphoreType.DMA,
            pltpu.SemaphoreType.DMA,
        ],
        compiler_params=pltpu.CompilerParams(
            needs_layout_passes=False, skip_device_barrier=True),
    )
    def gather_kernel(idx_hbm, zt_hbm, out_hbm, idx_v, tiles_v, cols_v,
                      sem0, sem1, sem2):
        wid = lax.axis_index("s") * nc + lax.axis_index("c")
        base = pl.multiple_of(wid * chunk, _LANES)
        pltpu.sync_copy(idx_hbm.at[pl.ds(base, chunk)], idx_v)

        iota = lax.broadcasted_iota(jnp.int32, (_WAVE,), 0)
        sems = (sem0, sem1, sem2)

        def fire(g, buf):
            idx16 = idx_v[pl.ds(g * _WAVE, _WAVE)]
            for k in range(_WAVE):
                lt = pl.multiple_of(
                    lax.shift_right_logical(idx16[k], 7) * _LANES, _LANES)
                pltpu.async_copy(
                    zt_hbm.at[:, pl.ds(lt, _LANES)],
                    tiles_v.at[:, pl.ds((buf * _WAVE + k) * _LANES, _LANES)],
                    sems[buf])

        def drain_extract(g, buf):
            pltpu.make_async_copy(
                zt_hbm.at[:, pl.ds(0, _WAVE * _LANES)],
                tiles_v.at[:, pl.ds(buf * _WAVE * _LANES, _WAVE * _LANES)],
                sems[buf]).wait()
            idx16 = idx_v[pl.ds(g * _WAVE, _WAVE)]
            col16 = (buf * _WAVE + iota) * _LANES + (idx16 & (_LANES - 1))
            for j in range(dim):
                row_j = jnp.full((_WAVE,), j, jnp.int32)
                vals = plsc.load_gather(tiles_v, [row_j, col16])
                cols_v[j, pl.ds(g * _WAVE, _WAVE)] = vals

        fire(0, 0)
        fire(1, 1)
        n_main = (n_waves // _NBUF) * _NBUF

        def trio_body(p, _):
            g = _NBUF * p
            for q in range(_NBUF):
                nxt = g + q + _NBUF - 1
                nxt_buf = (q + _NBUF - 1) % _NBUF

                @pl.when(nxt < n_waves)
                def _fire_next():
                    fire(nxt, nxt_buf)

                drain_extract(g + q, q)
            return _

        lax.fori_loop(0, n_main // _NBUF, trio_body, None)
        for g in range(n_main, n_waves):
            drain_extract(g, g % _NBUF)
        pltpu.sync_copy(cols_v, out_hbm.at[:, pl.ds(base, chunk)])

    return gather_kernel


def kernel(ixs, z):
    n, dim = z.shape
    batch = ixs.shape[0]
    out_t = _build(batch, dim)(ixs, z.T)
    return out_t.T
